# baseline (speedup 1.0000x reference)
import functools

import jax
import jax.numpy as jnp
from jax import lax
from jax.experimental import pallas as pl
from jax.experimental.pallas import tpu as pltpu
from jax.experimental.pallas import tpu_sc as plsc

N_HID = 32
B = 8
L = 50
LP = 56
N = B * L
NP = B * LP
E_CHUNK = 16


def _sc_mesh():
    return plsc.VectorSubcoreMesh(core_axis_name="c", subcore_axis_name="s")


NW = 32


def _sc_stage(x2d, src, dst, ew, zeros, embed_w, n_edges):
    n_gather = x2d.shape[0]
    n_chunks = n_edges // E_CHUNK
    per_w = (n_chunks + NW - 1) // NW

    @functools.partial(
        pl.kernel,
        mesh=_sc_mesh(),
        out_type=[
            jax.ShapeDtypeStruct((N, N_HID), jnp.float32),
            jax.ShapeDtypeStruct((NW, N, N_HID), jnp.float32),
        ],
        scratch_types=[
            pltpu.VMEM((n_gather, L), jnp.int32),
            pltpu.VMEM((N, N_HID), jnp.float32),
            pltpu.VMEM((n_edges,), jnp.int32),
            pltpu.VMEM((n_edges,), jnp.int32),
            pltpu.VMEM((n_edges,), jnp.float32),
            pltpu.VMEM((N, N_HID), jnp.float32),
            pltpu.SemaphoreType.DMA,
        ],
        compiler_params=pltpu.CompilerParams(
            use_tc_tiling_on_sc=False, needs_layout_passes=False),
    )
    def k(x_hbm, src_hbm, dst_hbm, ew_hbm, zero_hbm, table_hbm,
          h_out, a_out, x_v, h_v, src_v, dst_v, ew_v, agg_v, sem):
        wid = lax.axis_index("c") * 16 + lax.axis_index("s")

        pltpu.sync_copy(x_hbm, x_v)
        pltpu.sync_copy(src_hbm, src_v)
        pltpu.sync_copy(dst_hbm, dst_v)
        pltpu.sync_copy(ew_hbm, ew_v)
        pltpu.sync_copy(zero_hbm, agg_v)

        copies = []
        for g in range(n_gather):
            copies.append(
                pltpu.async_copy(
                    table_hbm.at[x_v.at[g]],
                    h_v.at[pl.ds(g * L, L)],
                    sem,
                )
            )
        for c in copies:
            c.wait()

        for t in range(per_w):
            kk = wid + NW * t

            @pl.when(kk < n_chunks)
            def _():
                s_idx = src_v[pl.ds(kk * E_CHUNK, E_CHUNK)]
                d_idx = dst_v[pl.ds(kk * E_CHUNK, E_CHUNK)]
                w = ew_v[pl.ds(kk * E_CHUNK, E_CHUNK)]
                for j in range(N_HID):
                    colj = jnp.full((E_CHUNK,), j, jnp.int32)
                    vals = plsc.load_gather(h_v, [s_idx, colj])
                    plsc.addupdate_scatter(agg_v, [d_idx, colj], w * vals)

        pltpu.sync_copy(agg_v, a_out.at[wid])

        @pl.when(wid == 0)
        def _():
            pltpu.sync_copy(h_v, h_out)

    return k(x2d, src, dst, ew, zeros, embed_w)


def _pre_body(h_ref, a0s_ref, ggc_ref, wiT_ref, whT_ref, bi_ref,
              bh_ref, W1t_ref, b1_ref, W2t_ref, b2_ref, Wtt_ref, bt_ref,
              qt_ref, qb_ref, W3t_ref, b3_ref, wall_ref):
    f32 = jnp.float32

    def mm(a, b):
        return jnp.dot(a, b, preferred_element_type=f32)

    h = h_ref[...]
    a0 = a0s_ref[0]
    for wix in range(1, NW):
        a0 = a0 + a0s_ref[wix]
    agg = mm(a0, ggc_ref[...])

    gi = mm(agg, wiT_ref[...]) + bi_ref[...]
    gh = mm(h, whT_ref[...]) + bh_ref[...]
    r = jax.nn.sigmoid(gi[:, 0:N_HID] + gh[:, 0:N_HID])
    zg = jax.nn.sigmoid(gi[:, N_HID:2 * N_HID] + gh[:, N_HID:2 * N_HID])
    ng = jnp.tanh(gi[:, 2 * N_HID:] + r * gh[:, 2 * N_HID:])
    v = (1.0 - zg) * ng + zg * h

    bb = lax.broadcasted_iota(jnp.int32, (B, N), 0)
    nn = lax.broadcasted_iota(jnp.int32, (B, N), 1)
    sel_lastT = (nn == bb * L + (L - 1)).astype(f32)
    segT = (nn // L == bb).astype(f32)
    nn2 = lax.broadcasted_iota(jnp.int32, (N, B), 0)
    bb2 = lax.broadcasted_iota(jnp.int32, (N, B), 1)
    seg = (nn2 // L == bb2).astype(f32)

    s_l = mm(sel_lastT, v)
    v_n = mm(seg, s_l)
    pre = jax.nn.sigmoid(mm(v_n, W1t_ref[...]) + b1_ref[...]
                         + mm(v, W2t_ref[...]) + b2_ref[...])
    alpha = mm(pre, qt_ref[...]) + qb_ref[...]
    s_g = mm(segT, alpha * v)
    base = mm(jnp.concatenate([s_l, s_g], axis=1), W3t_ref[...]) + b3_ref[...]

    rr = lax.broadcasted_iota(jnp.int32, (NP, N), 0)
    cc = lax.broadcasted_iota(jnp.int32, (NP, N), 1)
    tgt = (rr // LP) * L + rr % LP
    epad = ((rr % LP < L) & (cc == tgt)).astype(f32)
    vpad = mm(epad, v)
    tpad = mm(vpad, Wtt_ref[...]) + bt_ref[...]

    wall_ref[0:NP, :] = tpad
    wall_ref[NP:2 * NP, :] = vpad
    wall_ref[2 * NP:2 * NP + B, :] = base


def _big_body(w_ref, e_ref, o_ref):
    f32 = jnp.float32
    MT = lax.dot_general(w_ref[...].astype(jnp.bfloat16), e_ref[...],
                         (((1,), (1,)), ((), ())),
                         preferred_element_type=f32)
    rowmask = (lax.broadcasted_iota(jnp.int32, (LP, 1), 0) < L).astype(f32)
    rows = []
    for b in range(B):
        Ab = MT[b * LP:(b + 1) * LP, :]
        Pb = MT[NP + b * LP:NP + (b + 1) * LP, :]
        ex = jnp.exp(Ab) * rowmask
        num = jnp.sum(ex * Pb, axis=0, keepdims=True)
        den = jnp.sum(ex, axis=0, keepdims=True)
        rows.append(num / den)
    o_ref[...] = jnp.concatenate(rows, axis=0) + MT[2 * NP:2 * NP + B, :]


def _big(embed_w, wall, tile):
    vocab = embed_w.shape[0]
    grid = pl.cdiv(vocab, tile)
    return pl.pallas_call(
        _big_body,
        grid=(grid,),
        in_specs=[
            pl.BlockSpec((2 * NP + B, N_HID), lambda i: (0, 0)),
            pl.BlockSpec((tile, N_HID), lambda i: (i, 0)),
        ],
        out_specs=pl.BlockSpec((B, tile), lambda i: (0, i)),
        out_shape=jax.ShapeDtypeStruct((B, vocab), jnp.float32),
    )(wall, embed_w.astype(jnp.bfloat16))


def kernel(x, edge_index, edge_weight, batch, embed_w, ggc_w, gru_wi, gru_wh,
           gru_bi, gru_bh, W1_w, W1_b, W2_w, W2_b, Wt_w, Wt_b, q_w, q_b,
           W3_w, W3_b):
    n_edges = edge_index.shape[1]
    x2d = x.reshape(B, L)
    src = edge_index[0]
    dst = edge_index[1]
    zeros = jnp.zeros((N, N_HID), jnp.float32)

    h, a0s = _sc_stage(x2d, src, dst, edge_weight, zeros, embed_w, n_edges)

    wall = pl.pallas_call(
        _pre_body,
        out_shape=jax.ShapeDtypeStruct((2 * NP + B, N_HID), jnp.float32),
    )(
        h, a0s, ggc_w[0], gru_wi.T, gru_wh.T,
        gru_bi.reshape(1, 3 * N_HID), gru_bh.reshape(1, 3 * N_HID),
        W1_w.T, W1_b.reshape(1, N_HID), W2_w.T, W2_b.reshape(1, N_HID),
        Wt_w.T, Wt_b.reshape(1, N_HID), q_w.T, q_b.reshape(1, 1),
        W3_w.T, W3_b.reshape(1, N_HID),
    )

    return _big(embed_w, wall, tile=4096)


# speedup vs baseline: 1.0490x; 1.0490x over previous
import functools

import jax
import jax.numpy as jnp
from jax import lax
from jax.experimental import pallas as pl
from jax.experimental.pallas import tpu as pltpu
from jax.experimental.pallas import tpu_sc as plsc

N_HID = 32
B = 8
L = 50
LP = 56
N = B * L
NP = B * LP
E_CHUNK = 16


def _sc_mesh():
    return plsc.VectorSubcoreMesh(core_axis_name="c", subcore_axis_name="s")


NW = 32


def _sc_stage(x2d, src, dst, ew, zeros, embed_w, n_edges):
    n_gather = x2d.shape[0]
    n_chunks = n_edges // E_CHUNK
    per_w = (n_chunks + NW - 1) // NW

    @functools.partial(
        pl.kernel,
        mesh=_sc_mesh(),
        out_type=[
            jax.ShapeDtypeStruct((N, N_HID), jnp.float32),
            jax.ShapeDtypeStruct((NW, N, N_HID), jnp.float32),
        ],
        scratch_types=[
            pltpu.VMEM((n_gather, L), jnp.int32),
            pltpu.VMEM((N, N_HID), jnp.float32),
            pltpu.VMEM((n_edges,), jnp.int32),
            pltpu.VMEM((n_edges,), jnp.int32),
            pltpu.VMEM((n_edges,), jnp.float32),
            pltpu.VMEM((N, N_HID), jnp.float32),
            pltpu.SemaphoreType.DMA,
        ],
        compiler_params=pltpu.CompilerParams(
            use_tc_tiling_on_sc=False, needs_layout_passes=False),
    )
    def k(x_hbm, src_hbm, dst_hbm, ew_hbm, zero_hbm, table_hbm,
          h_out, a_out, x_v, h_v, src_v, dst_v, ew_v, agg_v, sem):
        wid = lax.axis_index("c") * 16 + lax.axis_index("s")

        pltpu.sync_copy(x_hbm, x_v)
        pltpu.sync_copy(src_hbm, src_v)
        pltpu.sync_copy(dst_hbm, dst_v)
        pltpu.sync_copy(ew_hbm, ew_v)
        pltpu.sync_copy(zero_hbm, agg_v)

        copies = []
        for g in range(n_gather):
            copies.append(
                pltpu.async_copy(
                    table_hbm.at[x_v.at[g]],
                    h_v.at[pl.ds(g * L, L)],
                    sem,
                )
            )
        for c in copies:
            c.wait()

        for t in range(per_w):
            kk = wid + NW * t

            @pl.when(kk < n_chunks)
            def _():
                s_idx = src_v[pl.ds(kk * E_CHUNK, E_CHUNK)]
                d_idx = dst_v[pl.ds(kk * E_CHUNK, E_CHUNK)]
                w = ew_v[pl.ds(kk * E_CHUNK, E_CHUNK)]
                for j in range(N_HID):
                    colj = jnp.full((E_CHUNK,), j, jnp.int32)
                    vals = plsc.load_gather(h_v, [s_idx, colj])
                    plsc.addupdate_scatter(agg_v, [d_idx, colj], w * vals)

        pltpu.sync_copy(agg_v, a_out.at[wid])

        @pl.when(wid == 0)
        def _():
            pltpu.sync_copy(h_v, h_out)

    return k(x2d, src, dst, ew, zeros, embed_w)


def _pre_body(h_ref, a0s_ref, ggc_ref, wiT_ref, whT_ref, bi_ref,
              bh_ref, W1t_ref, b1_ref, W2t_ref, b2_ref, Wtt_ref, bt_ref,
              qt_ref, qb_ref, W3t_ref, b3_ref, wall_ref):
    f32 = jnp.float32

    def mm(a, b):
        return jnp.dot(a, b, preferred_element_type=f32)

    h = h_ref[...]
    a0 = a0s_ref[0]
    for wix in range(1, NW):
        a0 = a0 + a0s_ref[wix]
    agg = mm(a0, ggc_ref[...])

    gi = mm(agg, wiT_ref[...]) + bi_ref[...]
    gh = mm(h, whT_ref[...]) + bh_ref[...]
    r = jax.nn.sigmoid(gi[:, 0:N_HID] + gh[:, 0:N_HID])
    zg = jax.nn.sigmoid(gi[:, N_HID:2 * N_HID] + gh[:, N_HID:2 * N_HID])
    ng = jnp.tanh(gi[:, 2 * N_HID:] + r * gh[:, 2 * N_HID:])
    v = (1.0 - zg) * ng + zg * h

    bb = lax.broadcasted_iota(jnp.int32, (B, N), 0)
    nn = lax.broadcasted_iota(jnp.int32, (B, N), 1)
    sel_lastT = (nn == bb * L + (L - 1)).astype(f32)
    segT = (nn // L == bb).astype(f32)
    nn2 = lax.broadcasted_iota(jnp.int32, (N, B), 0)
    bb2 = lax.broadcasted_iota(jnp.int32, (N, B), 1)
    seg = (nn2 // L == bb2).astype(f32)

    s_l = mm(sel_lastT, v)
    v_n = mm(seg, s_l)
    pre = jax.nn.sigmoid(mm(v_n, W1t_ref[...]) + b1_ref[...]
                         + mm(v, W2t_ref[...]) + b2_ref[...])
    alpha = mm(pre, qt_ref[...]) + qb_ref[...]
    s_g = mm(segT, alpha * v)
    base = mm(jnp.concatenate([s_l, s_g], axis=1), W3t_ref[...]) + b3_ref[...]

    rr = lax.broadcasted_iota(jnp.int32, (NP, N), 0)
    cc = lax.broadcasted_iota(jnp.int32, (NP, N), 1)
    tgt = (rr // LP) * L + rr % LP
    epad = ((rr % LP < L) & (cc == tgt)).astype(f32)
    vpad = mm(epad, v)
    tpad = mm(vpad, Wtt_ref[...]) + bt_ref[...]

    wall_ref[0:NP, :] = tpad
    wall_ref[NP:2 * NP, :] = vpad
    wall_ref[2 * NP:2 * NP + B, :] = base


def _big_body(w_ref, e_ref, o_ref):
    f32 = jnp.float32
    MT = lax.dot_general(w_ref[...].astype(jnp.bfloat16),
                         e_ref[...].astype(jnp.bfloat16),
                         (((1,), (1,)), ((), ())),
                         preferred_element_type=f32)
    rowmask = (lax.broadcasted_iota(jnp.int32, (LP, 1), 0) < L).astype(f32)
    rows = []
    for b in range(B):
        Ab = MT[b * LP:(b + 1) * LP, :]
        Pb = MT[NP + b * LP:NP + (b + 1) * LP, :]
        ex = jnp.exp(Ab) * rowmask
        num = jnp.sum(ex * Pb, axis=0, keepdims=True)
        den = jnp.sum(ex, axis=0, keepdims=True)
        rows.append(num / den)
    o_ref[...] = jnp.concatenate(rows, axis=0) + MT[2 * NP:2 * NP + B, :]


def _big(embed_w, wall, tile):
    vocab = embed_w.shape[0]
    grid = pl.cdiv(vocab, tile)
    return pl.pallas_call(
        _big_body,
        grid=(grid,),
        in_specs=[
            pl.BlockSpec((2 * NP + B, N_HID), lambda i: (0, 0)),
            pl.BlockSpec((tile, N_HID), lambda i: (i, 0)),
        ],
        out_specs=pl.BlockSpec((B, tile), lambda i: (0, i)),
        out_shape=jax.ShapeDtypeStruct((B, vocab), jnp.float32),
    )(wall, embed_w)


def kernel(x, edge_index, edge_weight, batch, embed_w, ggc_w, gru_wi, gru_wh,
           gru_bi, gru_bh, W1_w, W1_b, W2_w, W2_b, Wt_w, Wt_b, q_w, q_b,
           W3_w, W3_b):
    n_edges = edge_index.shape[1]
    x2d = x.reshape(B, L)
    src = edge_index[0]
    dst = edge_index[1]
    zeros = jnp.zeros((N, N_HID), jnp.float32)

    h, a0s = _sc_stage(x2d, src, dst, edge_weight, zeros, embed_w, n_edges)

    wall = pl.pallas_call(
        _pre_body,
        out_shape=jax.ShapeDtypeStruct((2 * NP + B, N_HID), jnp.float32),
    )(
        h, a0s, ggc_w[0], gru_wi.T, gru_wh.T,
        gru_bi.reshape(1, 3 * N_HID), gru_bh.reshape(1, 3 * N_HID),
        W1_w.T, W1_b.reshape(1, N_HID), W2_w.T, W2_b.reshape(1, N_HID),
        Wt_w.T, Wt_b.reshape(1, N_HID), q_w.T, q_b.reshape(1, 1),
        W3_w.T, W3_b.reshape(1, N_HID),
    )

    return _big(embed_w, wall, tile=4096)
